# baseline (speedup 1.0000x reference)
import functools

import jax
import jax.numpy as jnp
from jax import lax
from jax.experimental import pallas as pl
from jax.experimental.pallas import tpu as pltpu
from jax.experimental.pallas import tpu_sc as plsc

N = 10000
NP = 10240
E = 160000
EP = 163840
NC = 2
NS = 16
CW = 128
ROW_BLK = 1000

_MESH = dict(core_axis_name="c", subcore_axis_name="s",
             num_cores=NC, num_subcores=NS)


def _deg_body(dst2d, out, deg_sp, idx, ones_v, zbuf, bounce):
    c = lax.axis_index("c")
    s = lax.axis_index("s")
    for j in range(128 // 16):
        ones_v[pl.ds(j * 16, 16)] = jnp.ones((16,), jnp.float32)
    for j in range(640 // 16):
        zbuf[pl.ds(j * 16, 16)] = jnp.zeros((16,), jnp.float32)
    pltpu.sync_copy(zbuf, deg_sp.at[pl.ds(s * 640, 640)])
    plsc.subcore_barrier()
    w0 = (c * NS + s) * 40

    def macro(m, carry):
        row0 = w0 + m * 8
        pltpu.sync_copy(dst2d.at[pl.ds(row0, 8)], idx)
        for j in range(8):
            pltpu.sync_copy(ones_v, deg_sp.at[idx.at[j]], add=True)
        return carry

    lax.fori_loop(0, 5, macro, 0)
    plsc.subcore_barrier()

    @pl.when(s < 10)
    def _():
        pltpu.sync_copy(deg_sp.at[pl.ds(s * 1000, 1000)], bounce)
        pltpu.sync_copy(bounce, out.at[pl.ds(c * N + s * 1000, 1000)])


def _deg_call(dst2d):
    return pl.kernel(
        _deg_body,
        out_type=jax.ShapeDtypeStruct((2 * N,), jnp.float32),
        mesh=plsc.VectorSubcoreMesh(**_MESH),
        scratch_types=[
            pltpu.VMEM_SHARED((NP,), jnp.float32),
            pltpu.VMEM((8, 128), jnp.int32),
            pltpu.VMEM((128,), jnp.float32),
            pltpu.VMEM((640,), jnp.float32),
            pltpu.VMEM((1000,), jnp.float32),
        ],
    )(dst2d)


def _prop_body(C, y_hbm, src2d, dst2d, out, acc_sp, isrc, idst, rows, zbuf,
               sg0, sg1, sg2, sg3, ss0, ss1, ss2, ss3, si0, si1, sz):
    c = lax.axis_index("c")
    s = lax.axis_index("s")
    cpc = C // NC
    sg = [sg0, sg1, sg2, sg3]
    ss = [ss0, ss1, ss2, ss3]
    w0 = s * 80
    r0 = s * 640
    for i in range(64):
        for k in range(0, CW, 16):
            zbuf[i, pl.ds(k, 16)] = jnp.zeros((16,), jnp.float32)
    for cc in range(cpc):
        chunk = c * cpc + cc
        for p in range(10):
            pltpu.async_copy(zbuf, acc_sp.at[pl.ds(r0 + p * 64, 64)], sz)
        for p in range(10):
            pltpu.make_async_copy(zbuf, acc_sp.at[pl.ds(r0 + p * 64, 64)],
                                  sz).wait()
        pltpu.sync_copy(src2d.at[pl.ds(w0, 8)], isrc.at[0])
        pltpu.sync_copy(dst2d.at[pl.ds(w0, 8)], idst.at[0])
        plsc.subcore_barrier()

        def macro(m, carry):
            ib = m % 2
            nb = (m + 1) % 2

            @pl.when(m > 0)
            def _():
                pltpu.make_async_copy(
                    src2d.at[pl.ds(w0 + m * 8, 8)], isrc.at[ib], si0).wait()
                pltpu.make_async_copy(
                    dst2d.at[pl.ds(w0 + m * 8, 8)], idst.at[ib], si1).wait()

            @pl.when(m + 1 < 10)
            def _():
                pltpu.async_copy(
                    src2d.at[pl.ds(w0 + (m + 1) * 8, 8)], isrc.at[nb], si0)
                pltpu.async_copy(
                    dst2d.at[pl.ds(w0 + (m + 1) * 8, 8)], idst.at[nb], si1)

            def gather(j):
                return pltpu.async_copy(
                    y_hbm.at[chunk].at[isrc.at[ib, j // 2, j % 2]],
                    rows.at[j % 4], sg[j % 4])

            g = [gather(0), gather(1), None, None]
            sc = [None, None, None, None]
            for j in range(16):
                b = j % 4
                g[b].wait()
                sc[b] = pltpu.async_copy(
                    rows.at[b], acc_sp.at[idst.at[ib, j // 2, j % 2]],
                    ss[b], add=True)
                nj = j + 2
                if nj < 16:
                    nb2 = nj % 4
                    if sc[nb2] is not None:
                        sc[nb2].wait()
                    g[nb2] = gather(nj)
            for b in range(4):
                if sc[b] is not None:
                    sc[b].wait()
            return carry

        lax.fori_loop(0, 10, macro, 0)
        plsc.subcore_barrier()
        st = [None, None]
        for p in range(10):
            b = p % 2
            if st[b] is not None:
                st[b].wait()
            pltpu.sync_copy(acc_sp.at[pl.ds(r0 + p * 64, 64)], rows.at[b])
            st[b] = pltpu.async_copy(
                rows.at[b], out.at[chunk, pl.ds(r0 + p * 64, 64)], sg[b])
        st[0].wait()
        st[1].wait()


def _prop_call(C, y, src2d, dst2d):
    return pl.kernel(
        functools.partial(_prop_body, C),
        out_type=jax.ShapeDtypeStruct((C, NP, CW), jnp.float32),
        mesh=plsc.VectorSubcoreMesh(**_MESH),
        scratch_types=[
            pltpu.VMEM_SHARED((NP, CW), jnp.float32),
            pltpu.VMEM((2, 8, 2, 64), jnp.int32),
            pltpu.VMEM((2, 8, 2, 64), jnp.int32),
            pltpu.VMEM((4, 64, CW), jnp.float32),
            pltpu.VMEM((64, CW), jnp.float32),
        ] + [pltpu.SemaphoreType.DMA] * 11,
    )(y, src2d, dst2d)


def _prep_kernel(x_ref, dp_ref, y0_ref, dinv_ref):
    deg = dp_ref[:, 0] + dp_ref[:, 1] + 1.0
    dinv = lax.rsqrt(deg)
    y = x_ref[...] * dinv[:, None]
    y0_ref[0] = y[:, 0:CW]
    y0_ref[1] = y[:, CW:2 * CW]
    dinv_ref[...] = dinv[:, None]


def _prep_call(x, deg_part):
    g = N // ROW_BLK
    return pl.pallas_call(
        _prep_kernel,
        grid=(g,),
        in_specs=[
            pl.BlockSpec((ROW_BLK, 2 * CW), lambda i: (i, 0)),
            pl.BlockSpec((ROW_BLK, 2), lambda i: (i, 0)),
        ],
        out_specs=[
            pl.BlockSpec((2, ROW_BLK, CW), lambda i: (0, i, 0)),
            pl.BlockSpec((ROW_BLK, 1), lambda i: (i, 0)),
        ],
        out_shape=[
            jax.ShapeDtypeStruct((2, NP, CW), jnp.float32),
            jax.ShapeDtypeStruct((N, 1), jnp.float32),
        ],
    )(x, deg_part.reshape(2, N).T)


def _layer_kernel(c_in, c_out, acc_ref, y_ref, dinv_ref, w_ref, b_ref,
                  out_ref):
    m = None
    for ci in range(c_in):
        p = lax.dot_general(acc_ref[ci] + y_ref[ci],
                            w_ref[:, ci * CW:(ci + 1) * CW],
                            (((1,), (1,)), ((), ())),
                            preferred_element_type=jnp.float32)
        m = p if m is None else m + p
    dinv = dinv_ref[...]
    z = jnp.maximum(m * dinv + b_ref[...], 0.0)
    y2 = z * dinv
    for co in range(c_out):
        out_ref[co] = y2[:, co * CW:(co + 1) * CW]


def _layer_call(acc, y, dinv, w, b, c_in, c_out):
    g = N // ROW_BLK
    f_out = w.shape[0]
    return pl.pallas_call(
        functools.partial(_layer_kernel, c_in, c_out),
        grid=(g,),
        in_specs=[
            pl.BlockSpec((c_in, ROW_BLK, CW), lambda i: (0, i, 0)),
            pl.BlockSpec((c_in, ROW_BLK, CW), lambda i: (0, i, 0)),
            pl.BlockSpec((ROW_BLK, 1), lambda i: (i, 0)),
            pl.BlockSpec(w.shape, lambda i: (0, 0)),
            pl.BlockSpec((1, f_out), lambda i: (0, 0)),
        ],
        out_specs=pl.BlockSpec((c_out, ROW_BLK, CW), lambda i: (0, i, 0)),
        out_shape=jax.ShapeDtypeStruct((c_out, NP, CW), jnp.float32),
    )(acc, y, dinv, w, b.reshape(1, f_out))


def _final_kernel(acc_ref, y_ref, dinv_ref, w3_ref, b3_ref, wl1_ref, bl1_ref,
                  wl2_ref, bl2_ref, wl3_ref, bl3_ref, out_ref):
    m = None
    for ci in range(4):
        p = lax.dot_general(acc_ref[ci] + y_ref[ci],
                            w3_ref[:, ci * CW:(ci + 1) * CW],
                            (((1,), (1,)), ((), ())),
                            preferred_element_type=jnp.float32)
        m = p if m is None else m + p
    h = jnp.maximum(m * dinv_ref[...] + b3_ref[...], 0.0)
    h = jnp.maximum(
        lax.dot_general(h, wl1_ref[...], (((1,), (1,)), ((), ())),
                        preferred_element_type=jnp.float32) + bl1_ref[...], 0.0)
    h = jnp.maximum(
        lax.dot_general(h, wl2_ref[...], (((1,), (1,)), ((), ())),
                        preferred_element_type=jnp.float32) + bl2_ref[...], 0.0)
    out_ref[...] = lax.dot_general(
        h, wl3_ref[...], (((1,), (1,)), ((), ())),
        preferred_element_type=jnp.float32) + bl3_ref[...]


def _final_call(acc, y, dinv, w3, b3, wl1, bl1, wl2, bl2, wl3, bl3):
    g = N // ROW_BLK
    full = lambda a: pl.BlockSpec(a.shape, lambda i: tuple(0 for _ in a.shape))
    return pl.pallas_call(
        _final_kernel,
        grid=(g,),
        in_specs=[
            pl.BlockSpec((4, ROW_BLK, CW), lambda i: (0, i, 0)),
            pl.BlockSpec((4, ROW_BLK, CW), lambda i: (0, i, 0)),
            pl.BlockSpec((ROW_BLK, 1), lambda i: (i, 0)),
            full(w3), pl.BlockSpec((1, 1024), lambda i: (0, 0)),
            full(wl1), pl.BlockSpec((1, 512), lambda i: (0, 0)),
            full(wl2), pl.BlockSpec((1, 256), lambda i: (0, 0)),
            full(wl3), pl.BlockSpec((1, 256), lambda i: (0, 0)),
        ],
        out_specs=pl.BlockSpec((ROW_BLK, 256), lambda i: (i, 0)),
        out_shape=jax.ShapeDtypeStruct((N, 256), jnp.float32),
    )(acc, y, dinv, w3, b3.reshape(1, -1), wl1, bl1.reshape(1, -1),
      wl2, bl2.reshape(1, -1), wl3, bl3.reshape(1, -1))


def kernel(x, edge_index, W1, b1, W2, b2, W3, b3, Wl1, bl1, Wl2, bl2, Wl3, bl3):
    src = edge_index[0].astype(jnp.int32)
    dst = edge_index[1].astype(jnp.int32)
    pad = jnp.arange(EP - E, dtype=jnp.int32)
    src2d = jnp.concatenate([src, pad % N]).reshape(EP // 128, 2, 64)
    dst3d = jnp.concatenate([dst, N + (pad % 16)]).reshape(EP // 128, 2, 64)
    dst2d = jnp.concatenate([dst, N + (pad % 16)]).reshape(EP // 128, 128)

    deg_part = _deg_call(dst2d)
    y0, dinv = _prep_call(x, deg_part)
    acc1 = _prop_call(2, y0, src2d, dst3d)
    y1 = _layer_call(acc1, y0, dinv, W1, b1, 2, 4)
    acc2 = _prop_call(4, y1, src2d, dst3d)
    y2 = _layer_call(acc2, y1, dinv, W2, b2, 4, 4)
    acc3 = _prop_call(4, y2, src2d, dst3d)
    return _final_call(acc3, y2, dinv, W3, b3, Wl1, bl1, Wl2, bl2, Wl3, bl3)


# speedup vs baseline: 1.0666x; 1.0666x over previous
import functools

import jax
import jax.numpy as jnp
from jax import lax
from jax.experimental import pallas as pl
from jax.experimental.pallas import tpu as pltpu
from jax.experimental.pallas import tpu_sc as plsc

N = 10000
NP = 10240
E = 160000
EP = 163840
NC = 2
NS = 16
CW = 128
ROW_BLK = 1000

_MESH = dict(core_axis_name="c", subcore_axis_name="s",
             num_cores=NC, num_subcores=NS)


def _deg_body(dst2d, out, deg_sp, idx, ones_v, zbuf, bounce):
    c = lax.axis_index("c")
    s = lax.axis_index("s")
    for j in range(128 // 16):
        ones_v[pl.ds(j * 16, 16)] = jnp.ones((16,), jnp.float32)
    for j in range(640 // 16):
        zbuf[pl.ds(j * 16, 16)] = jnp.zeros((16,), jnp.float32)
    pltpu.sync_copy(zbuf, deg_sp.at[pl.ds(s * 640, 640)])
    plsc.subcore_barrier()
    w0 = (c * NS + s) * 40

    def macro(m, carry):
        row0 = w0 + m * 8
        pltpu.sync_copy(dst2d.at[pl.ds(row0, 8)], idx)
        for j in range(8):
            pltpu.sync_copy(ones_v, deg_sp.at[idx.at[j]], add=True)
        return carry

    lax.fori_loop(0, 5, macro, 0)
    plsc.subcore_barrier()

    @pl.when(s < 10)
    def _():
        pltpu.sync_copy(deg_sp.at[pl.ds(s * 1000, 1000)], bounce)
        pltpu.sync_copy(bounce, out.at[pl.ds(c * N + s * 1000, 1000)])


def _deg_call(dst2d):
    return pl.kernel(
        _deg_body,
        out_type=jax.ShapeDtypeStruct((2 * N,), jnp.float32),
        mesh=plsc.VectorSubcoreMesh(**_MESH),
        scratch_types=[
            pltpu.VMEM_SHARED((NP,), jnp.float32),
            pltpu.VMEM((8, 128), jnp.int32),
            pltpu.VMEM((128,), jnp.float32),
            pltpu.VMEM((640,), jnp.float32),
            pltpu.VMEM((1000,), jnp.float32),
        ],
    )(dst2d)


def _prop_body(C, y_hbm, src2d, dst2d, out, acc_sp, isrc, idst, rows, zbuf,
               sg0, sg1, sg2, sg3, ss0, ss1, ss2, ss3, si0, si1, sz):
    c = lax.axis_index("c")
    s = lax.axis_index("s")
    cpc = C // NC
    sg = [sg0, sg1, sg2, sg3]
    ss = [ss0, ss1, ss2, ss3]
    w0 = s * 80
    r0 = s * 640
    for i in range(64):
        for k in range(0, CW, 16):
            zbuf[i, pl.ds(k, 16)] = jnp.zeros((16,), jnp.float32)
    for cc in range(cpc):
        chunk = c * cpc + cc
        for p in range(10):
            pltpu.async_copy(zbuf, acc_sp.at[pl.ds(r0 + p * 64, 64)], sz)
        for p in range(10):
            pltpu.make_async_copy(zbuf, acc_sp.at[pl.ds(r0 + p * 64, 64)],
                                  sz).wait()
        pltpu.sync_copy(src2d.at[pl.ds(w0, 8)], isrc.at[0])
        pltpu.sync_copy(dst2d.at[pl.ds(w0, 8)], idst.at[0])
        plsc.subcore_barrier()

        def macro(m, carry):
            ib = m % 2
            nb = (m + 1) % 2

            @pl.when(m > 0)
            def _():
                pltpu.make_async_copy(
                    src2d.at[pl.ds(w0 + m * 8, 8)], isrc.at[ib], si0).wait()
                pltpu.make_async_copy(
                    dst2d.at[pl.ds(w0 + m * 8, 8)], idst.at[ib], si1).wait()

            @pl.when(m + 1 < 10)
            def _():
                pltpu.async_copy(
                    src2d.at[pl.ds(w0 + (m + 1) * 8, 8)], isrc.at[nb], si0)
                pltpu.async_copy(
                    dst2d.at[pl.ds(w0 + (m + 1) * 8, 8)], idst.at[nb], si1)

            def gather(j):
                return pltpu.async_copy(
                    y_hbm.at[chunk].at[isrc.at[ib, j]], rows.at[j % 2],
                    sg[j % 2])

            g = [gather(0), gather(1)]
            sc = [None, None]
            for j in range(8):
                b = j % 2
                g[b].wait()
                sc[b] = pltpu.async_copy(
                    rows.at[b], acc_sp.at[idst.at[ib, j]], ss[b], add=True)
                if j + 2 < 8:
                    sc[b].wait()
                    g[b] = gather(j + 2)
            sc[0].wait()
            sc[1].wait()
            return carry

        lax.fori_loop(0, 10, macro, 0)
        plsc.subcore_barrier()
        st = [None, None]
        for p in range(5):
            b = p % 2
            if st[b] is not None:
                st[b].wait()
            pltpu.sync_copy(acc_sp.at[pl.ds(r0 + p * 128, 128)], rows.at[b])
            st[b] = pltpu.async_copy(
                rows.at[b], out.at[chunk, pl.ds(r0 + p * 128, 128)], sg[b])
        st[0].wait()
        st[1].wait()


def _prop_call(C, y, src2d, dst2d):
    return pl.kernel(
        functools.partial(_prop_body, C),
        out_type=jax.ShapeDtypeStruct((C, NP, CW), jnp.float32),
        mesh=plsc.VectorSubcoreMesh(**_MESH),
        scratch_types=[
            pltpu.VMEM_SHARED((NP, CW), jnp.float32),
            pltpu.VMEM((2, 8, 128), jnp.int32),
            pltpu.VMEM((2, 8, 128), jnp.int32),
            pltpu.VMEM((2, 128, CW), jnp.float32),
            pltpu.VMEM((64, CW), jnp.float32),
        ] + [pltpu.SemaphoreType.DMA] * 11,
    )(y, src2d, dst2d)


def _prep_kernel(x_ref, dp_ref, y0_ref, dinv_ref):
    deg = dp_ref[:, 0] + dp_ref[:, 1] + 1.0
    dinv = lax.rsqrt(deg)
    y = x_ref[...] * dinv[:, None]
    y0_ref[0] = y[:, 0:CW]
    y0_ref[1] = y[:, CW:2 * CW]
    dinv_ref[...] = dinv[:, None]


def _prep_call(x, deg_part):
    g = N // ROW_BLK
    return pl.pallas_call(
        _prep_kernel,
        grid=(g,),
        in_specs=[
            pl.BlockSpec((ROW_BLK, 2 * CW), lambda i: (i, 0)),
            pl.BlockSpec((ROW_BLK, 2), lambda i: (i, 0)),
        ],
        out_specs=[
            pl.BlockSpec((2, ROW_BLK, CW), lambda i: (0, i, 0)),
            pl.BlockSpec((ROW_BLK, 1), lambda i: (i, 0)),
        ],
        out_shape=[
            jax.ShapeDtypeStruct((2, NP, CW), jnp.float32),
            jax.ShapeDtypeStruct((N, 1), jnp.float32),
        ],
    )(x, deg_part.reshape(2, N).T)


def _layer_kernel(c_in, c_out, acc_ref, y_ref, dinv_ref, w_ref, b_ref,
                  out_ref):
    m = None
    for ci in range(c_in):
        p = lax.dot_general(acc_ref[ci] + y_ref[ci],
                            w_ref[:, ci * CW:(ci + 1) * CW],
                            (((1,), (1,)), ((), ())),
                            preferred_element_type=jnp.float32)
        m = p if m is None else m + p
    dinv = dinv_ref[...]
    z = jnp.maximum(m * dinv + b_ref[...], 0.0)
    y2 = z * dinv
    for co in range(c_out):
        out_ref[co] = y2[:, co * CW:(co + 1) * CW]


def _layer_call(acc, y, dinv, w, b, c_in, c_out):
    g = N // ROW_BLK
    f_out = w.shape[0]
    return pl.pallas_call(
        functools.partial(_layer_kernel, c_in, c_out),
        grid=(g,),
        in_specs=[
            pl.BlockSpec((c_in, ROW_BLK, CW), lambda i: (0, i, 0)),
            pl.BlockSpec((c_in, ROW_BLK, CW), lambda i: (0, i, 0)),
            pl.BlockSpec((ROW_BLK, 1), lambda i: (i, 0)),
            pl.BlockSpec(w.shape, lambda i: (0, 0)),
            pl.BlockSpec((1, f_out), lambda i: (0, 0)),
        ],
        out_specs=pl.BlockSpec((c_out, ROW_BLK, CW), lambda i: (0, i, 0)),
        out_shape=jax.ShapeDtypeStruct((c_out, NP, CW), jnp.float32),
    )(acc, y, dinv, w, b.reshape(1, f_out))


def _final_kernel(acc_ref, y_ref, dinv_ref, w3_ref, b3_ref, wl1_ref, bl1_ref,
                  wl2_ref, bl2_ref, wl3_ref, bl3_ref, out_ref):
    m = None
    for ci in range(4):
        p = lax.dot_general(acc_ref[ci] + y_ref[ci],
                            w3_ref[:, ci * CW:(ci + 1) * CW],
                            (((1,), (1,)), ((), ())),
                            preferred_element_type=jnp.float32)
        m = p if m is None else m + p
    h = jnp.maximum(m * dinv_ref[...] + b3_ref[...], 0.0)
    h = jnp.maximum(
        lax.dot_general(h, wl1_ref[...], (((1,), (1,)), ((), ())),
                        preferred_element_type=jnp.float32) + bl1_ref[...], 0.0)
    h = jnp.maximum(
        lax.dot_general(h, wl2_ref[...], (((1,), (1,)), ((), ())),
                        preferred_element_type=jnp.float32) + bl2_ref[...], 0.0)
    out_ref[...] = lax.dot_general(
        h, wl3_ref[...], (((1,), (1,)), ((), ())),
        preferred_element_type=jnp.float32) + bl3_ref[...]


def _final_call(acc, y, dinv, w3, b3, wl1, bl1, wl2, bl2, wl3, bl3):
    g = N // ROW_BLK
    full = lambda a: pl.BlockSpec(a.shape, lambda i: tuple(0 for _ in a.shape))
    return pl.pallas_call(
        _final_kernel,
        grid=(g,),
        in_specs=[
            pl.BlockSpec((4, ROW_BLK, CW), lambda i: (0, i, 0)),
            pl.BlockSpec((4, ROW_BLK, CW), lambda i: (0, i, 0)),
            pl.BlockSpec((ROW_BLK, 1), lambda i: (i, 0)),
            full(w3), pl.BlockSpec((1, 1024), lambda i: (0, 0)),
            full(wl1), pl.BlockSpec((1, 512), lambda i: (0, 0)),
            full(wl2), pl.BlockSpec((1, 256), lambda i: (0, 0)),
            full(wl3), pl.BlockSpec((1, 256), lambda i: (0, 0)),
        ],
        out_specs=pl.BlockSpec((ROW_BLK, 256), lambda i: (i, 0)),
        out_shape=jax.ShapeDtypeStruct((N, 256), jnp.float32),
    )(acc, y, dinv, w3, b3.reshape(1, -1), wl1, bl1.reshape(1, -1),
      wl2, bl2.reshape(1, -1), wl3, bl3.reshape(1, -1))


def kernel(x, edge_index, W1, b1, W2, b2, W3, b3, Wl1, bl1, Wl2, bl2, Wl3, bl3):
    src = edge_index[0].astype(jnp.int32)
    dst = edge_index[1].astype(jnp.int32)
    pad = jnp.arange(EP - E, dtype=jnp.int32)
    src2d = jnp.concatenate([src, pad % N]).reshape(EP // 128, 128)
    dst2d = jnp.concatenate([dst, N + (pad % 16)]).reshape(EP // 128, 128)

    deg_part = _deg_call(dst2d)
    y0, dinv = _prep_call(x, deg_part)
    acc1 = _prop_call(2, y0, src2d, dst2d)
    y1 = _layer_call(acc1, y0, dinv, W1, b1, 2, 4)
    acc2 = _prop_call(4, y1, src2d, dst2d)
    y2 = _layer_call(acc2, y1, dinv, W2, b2, 4, 4)
    acc3 = _prop_call(4, y2, src2d, dst2d)
    return _final_call(acc3, y2, dinv, W3, b3, Wl1, bl1, Wl2, bl2, Wl3, bl3)


# speedup vs baseline: 1.1548x; 1.0827x over previous
import functools

import jax
import jax.numpy as jnp
from jax import lax
from jax.experimental import pallas as pl
from jax.experimental.pallas import tpu as pltpu
from jax.experimental.pallas import tpu_sc as plsc

N = 10000
NP = 10240
E = 160000
EP = 163840
NC = 2
NS = 16
CW = 128
ROW_BLK = 1000

_MESH = dict(core_axis_name="c", subcore_axis_name="s",
             num_cores=NC, num_subcores=NS)


def _deg_body(dst2d, out, deg_sp, idx, ones_v, zbuf, bounce):
    c = lax.axis_index("c")
    s = lax.axis_index("s")
    for j in range(128 // 16):
        ones_v[pl.ds(j * 16, 16)] = jnp.ones((16,), jnp.float32)
    for j in range(640 // 16):
        zbuf[pl.ds(j * 16, 16)] = jnp.zeros((16,), jnp.float32)
    pltpu.sync_copy(zbuf, deg_sp.at[pl.ds(s * 640, 640)])
    plsc.subcore_barrier()
    w0 = (c * NS + s) * 40

    def macro(m, carry):
        row0 = w0 + m * 8
        pltpu.sync_copy(dst2d.at[pl.ds(row0, 8)], idx)
        for j in range(8):
            pltpu.sync_copy(ones_v, deg_sp.at[idx.at[j]], add=True)
        return carry

    lax.fori_loop(0, 5, macro, 0)
    plsc.subcore_barrier()

    @pl.when(s < 10)
    def _():
        pltpu.sync_copy(deg_sp.at[pl.ds(s * 1000, 1000)], bounce)
        pltpu.sync_copy(bounce, out.at[pl.ds(c * N + s * 1000, 1000)])


def _deg_call(dst2d):
    return pl.kernel(
        _deg_body,
        out_type=jax.ShapeDtypeStruct((2 * N,), jnp.float32),
        mesh=plsc.VectorSubcoreMesh(**_MESH),
        scratch_types=[
            pltpu.VMEM_SHARED((NP,), jnp.float32),
            pltpu.VMEM((8, 128), jnp.int32),
            pltpu.VMEM((128,), jnp.float32),
            pltpu.VMEM((640,), jnp.float32),
            pltpu.VMEM((1000,), jnp.float32),
        ],
    )(dst2d)


def _prop_body(C, y_hbm, src2d, dst2d, out, acc_sp, isrc, idst, rows, zbuf,
               sg0, sg1, sg2, sg3, ss0, ss1, ss2, ss3, si0, si1, sz):
    c = lax.axis_index("c")
    s = lax.axis_index("s")
    cpc = C // NC
    sg = [sg0, sg1, sg2, sg3]
    ss = [ss0, ss1, ss2, ss3]
    w0 = s * 80
    r0 = s * 640
    for i in range(64):
        for k in range(0, CW, 16):
            zbuf[i, pl.ds(k, 16)] = jnp.zeros((16,), jnp.float32)
    for cc in range(cpc):
        chunk = c * cpc + cc
        for p in range(10):
            pltpu.async_copy(zbuf, acc_sp.at[pl.ds(r0 + p * 64, 64)], sz)
        for p in range(10):
            pltpu.make_async_copy(zbuf, acc_sp.at[pl.ds(r0 + p * 64, 64)],
                                  sz).wait()
        pltpu.sync_copy(src2d.at[pl.ds(w0, 8)], isrc.at[0])
        pltpu.sync_copy(dst2d.at[pl.ds(w0, 8)], idst.at[0])
        plsc.subcore_barrier()

        def gather(w):
            return pltpu.async_copy(
                y_hbm.at[chunk].at[isrc.at[(w // 8) % 2, w % 8]],
                rows.at[w % 2], sg[w % 2])

        ih = [None, None]
        g = [gather(0), gather(1)]
        sc = [None, None]
        for w in range(80):
            m, j, b = w // 8, w % 8, w % 2
            if j == 4 and m + 1 < 10:
                nb = (m + 1) % 2
                ih[0] = pltpu.async_copy(
                    src2d.at[pl.ds(w0 + (m + 1) * 8, 8)], isrc.at[nb], si0)
                ih[1] = pltpu.async_copy(
                    dst2d.at[pl.ds(w0 + (m + 1) * 8, 8)], idst.at[nb], si1)
            g[b].wait()
            sc[b] = pltpu.async_copy(
                rows.at[b], acc_sp.at[idst.at[(m % 2), j]], ss[b], add=True)
            nw = w + 2
            if nw < 80:
                if nw % 8 == 0:
                    ih[0].wait()
                    ih[1].wait()
                sc[b].wait()
                g[b] = gather(nw)
        sc[0].wait()
        sc[1].wait()
        plsc.subcore_barrier()
        st = [None, None]
        for p in range(5):
            b = p % 2
            if st[b] is not None:
                st[b].wait()
            pltpu.sync_copy(acc_sp.at[pl.ds(r0 + p * 128, 128)], rows.at[b])
            st[b] = pltpu.async_copy(
                rows.at[b], out.at[chunk, pl.ds(r0 + p * 128, 128)], sg[b])
        st[0].wait()
        st[1].wait()


def _prop_call(C, y, src2d, dst2d):
    return pl.kernel(
        functools.partial(_prop_body, C),
        out_type=jax.ShapeDtypeStruct((C, NP, CW), jnp.float32),
        mesh=plsc.VectorSubcoreMesh(**_MESH),
        scratch_types=[
            pltpu.VMEM_SHARED((NP, CW), jnp.float32),
            pltpu.VMEM((2, 8, 128), jnp.int32),
            pltpu.VMEM((2, 8, 128), jnp.int32),
            pltpu.VMEM((2, 128, CW), jnp.float32),
            pltpu.VMEM((64, CW), jnp.float32),
        ] + [pltpu.SemaphoreType.DMA] * 11,
    )(y, src2d, dst2d)


def _prep_kernel(x_ref, dp_ref, y0_ref, dinv_ref):
    deg = dp_ref[:, 0] + dp_ref[:, 1] + 1.0
    dinv = lax.rsqrt(deg)
    y = x_ref[...] * dinv[:, None]
    y0_ref[0] = y[:, 0:CW]
    y0_ref[1] = y[:, CW:2 * CW]
    dinv_ref[...] = dinv[:, None]


def _prep_call(x, deg_part):
    g = N // ROW_BLK
    return pl.pallas_call(
        _prep_kernel,
        grid=(g,),
        in_specs=[
            pl.BlockSpec((ROW_BLK, 2 * CW), lambda i: (i, 0)),
            pl.BlockSpec((ROW_BLK, 2), lambda i: (i, 0)),
        ],
        out_specs=[
            pl.BlockSpec((2, ROW_BLK, CW), lambda i: (0, i, 0)),
            pl.BlockSpec((ROW_BLK, 1), lambda i: (i, 0)),
        ],
        out_shape=[
            jax.ShapeDtypeStruct((2, NP, CW), jnp.float32),
            jax.ShapeDtypeStruct((N, 1), jnp.float32),
        ],
    )(x, deg_part.reshape(2, N).T)


def _layer_kernel(c_in, c_out, acc_ref, y_ref, dinv_ref, w_ref, b_ref,
                  out_ref):
    m = None
    for ci in range(c_in):
        p = lax.dot_general(acc_ref[ci] + y_ref[ci],
                            w_ref[:, ci * CW:(ci + 1) * CW],
                            (((1,), (1,)), ((), ())),
                            preferred_element_type=jnp.float32)
        m = p if m is None else m + p
    dinv = dinv_ref[...]
    z = jnp.maximum(m * dinv + b_ref[...], 0.0)
    y2 = z * dinv
    for co in range(c_out):
        out_ref[co] = y2[:, co * CW:(co + 1) * CW]


def _layer_call(acc, y, dinv, w, b, c_in, c_out):
    g = N // ROW_BLK
    f_out = w.shape[0]
    return pl.pallas_call(
        functools.partial(_layer_kernel, c_in, c_out),
        grid=(g,),
        in_specs=[
            pl.BlockSpec((c_in, ROW_BLK, CW), lambda i: (0, i, 0)),
            pl.BlockSpec((c_in, ROW_BLK, CW), lambda i: (0, i, 0)),
            pl.BlockSpec((ROW_BLK, 1), lambda i: (i, 0)),
            pl.BlockSpec(w.shape, lambda i: (0, 0)),
            pl.BlockSpec((1, f_out), lambda i: (0, 0)),
        ],
        out_specs=pl.BlockSpec((c_out, ROW_BLK, CW), lambda i: (0, i, 0)),
        out_shape=jax.ShapeDtypeStruct((c_out, NP, CW), jnp.float32),
    )(acc, y, dinv, w, b.reshape(1, f_out))


def _final_kernel(acc_ref, y_ref, dinv_ref, w3_ref, b3_ref, wl1_ref, bl1_ref,
                  wl2_ref, bl2_ref, wl3_ref, bl3_ref, out_ref):
    m = None
    for ci in range(4):
        p = lax.dot_general(acc_ref[ci] + y_ref[ci],
                            w3_ref[:, ci * CW:(ci + 1) * CW],
                            (((1,), (1,)), ((), ())),
                            preferred_element_type=jnp.float32)
        m = p if m is None else m + p
    h = jnp.maximum(m * dinv_ref[...] + b3_ref[...], 0.0)
    h = jnp.maximum(
        lax.dot_general(h, wl1_ref[...], (((1,), (1,)), ((), ())),
                        preferred_element_type=jnp.float32) + bl1_ref[...], 0.0)
    h = jnp.maximum(
        lax.dot_general(h, wl2_ref[...], (((1,), (1,)), ((), ())),
                        preferred_element_type=jnp.float32) + bl2_ref[...], 0.0)
    out_ref[...] = lax.dot_general(
        h, wl3_ref[...], (((1,), (1,)), ((), ())),
        preferred_element_type=jnp.float32) + bl3_ref[...]


def _final_call(acc, y, dinv, w3, b3, wl1, bl1, wl2, bl2, wl3, bl3):
    g = N // ROW_BLK
    full = lambda a: pl.BlockSpec(a.shape, lambda i: tuple(0 for _ in a.shape))
    return pl.pallas_call(
        _final_kernel,
        grid=(g,),
        in_specs=[
            pl.BlockSpec((4, ROW_BLK, CW), lambda i: (0, i, 0)),
            pl.BlockSpec((4, ROW_BLK, CW), lambda i: (0, i, 0)),
            pl.BlockSpec((ROW_BLK, 1), lambda i: (i, 0)),
            full(w3), pl.BlockSpec((1, 1024), lambda i: (0, 0)),
            full(wl1), pl.BlockSpec((1, 512), lambda i: (0, 0)),
            full(wl2), pl.BlockSpec((1, 256), lambda i: (0, 0)),
            full(wl3), pl.BlockSpec((1, 256), lambda i: (0, 0)),
        ],
        out_specs=pl.BlockSpec((ROW_BLK, 256), lambda i: (i, 0)),
        out_shape=jax.ShapeDtypeStruct((N, 256), jnp.float32),
    )(acc, y, dinv, w3, b3.reshape(1, -1), wl1, bl1.reshape(1, -1),
      wl2, bl2.reshape(1, -1), wl3, bl3.reshape(1, -1))


def kernel(x, edge_index, W1, b1, W2, b2, W3, b3, Wl1, bl1, Wl2, bl2, Wl3, bl3):
    src = edge_index[0].astype(jnp.int32)
    dst = edge_index[1].astype(jnp.int32)
    pad = jnp.arange(EP - E, dtype=jnp.int32)
    src2d = jnp.concatenate([src, pad % N]).reshape(EP // 128, 128)
    dst2d = jnp.concatenate([dst, N + (pad % 16)]).reshape(EP // 128, 128)

    deg_part = _deg_call(dst2d)
    y0, dinv = _prep_call(x, deg_part)
    acc1 = _prop_call(2, y0, src2d, dst2d)
    y1 = _layer_call(acc1, y0, dinv, W1, b1, 2, 4)
    acc2 = _prop_call(4, y1, src2d, dst2d)
    y2 = _layer_call(acc2, y1, dinv, W2, b2, 4, 4)
    acc3 = _prop_call(4, y2, src2d, dst2d)
    return _final_call(acc3, y2, dinv, W3, b3, Wl1, bl1, Wl2, bl2, Wl3, bl3)
